# baseline (speedup 1.0000x reference)
import dataclasses
import functools

import jax
import jax.numpy as jnp
from jax import lax
from jax.experimental import pallas as pl
from jax.experimental.pallas import tpu as pltpu
from jax.experimental.pallas import tpu_sc as plsc

NBINS = 8192
LOSS_EPS = 1e-08
NC = 2
NS = 16
L = 16
NW = NC * NS
BLK_ROWS = 8


def _sc_histogram(inp):
    b, r, c = inp.shape
    mesh = plsc.VectorSubcoreMesh(core_axis_name="c", subcore_axis_name="s")
    cp = pltpu.CompilerParams(use_tc_tiling_on_sc=True)
    if "needs_layout_passes" in pltpu.CompilerParams.__dataclass_fields__:
        cp = dataclasses.replace(cp, needs_layout_passes=False)

    @functools.partial(
        pl.kernel,
        out_type=jax.ShapeDtypeStruct((NW, NBINS), jnp.int32),
        mesh=mesh,
        scratch_types=[pltpu.VMEM((NBINS,), jnp.int32)],
        compiler_params=cp,
    )
    def hist_kernel(idx_hbm, out_hbm, hist_v):
        wid = lax.axis_index("s") * NC + lax.axis_index("c")

        @pl.loop(0, NBINS, step=L)
        def _(i):
            hist_v[pl.ds(i, L)] = jnp.zeros((L,), jnp.int32)

        ones = jnp.ones((L,), jnp.int32)

        def body(idx_vmem):
            @pl.loop(0, BLK_ROWS, step=1)
            def _(row):
                @pl.loop(0, c, step=16 * L)
                def _(col):
                    xs = [
                        idx_vmem[0, row, pl.ds(col + u * L, L)]
                        for u in range(16)
                    ]
                    for x in xs:
                        plsc.addupdate_scatter(hist_v, [x], ones)

        pltpu.emit_pipeline(
            body,
            grid=(b, r // BLK_ROWS),
            in_specs=[
                pl.BlockSpec((1, BLK_ROWS, c), lambda i, j: (i, j, 0))
            ],
            out_specs=[],
            core_axis_name=("c", "s"),
            dimension_semantics=(pltpu.PARALLEL, pltpu.PARALLEL),
        )(idx_hbm)

        pltpu.sync_copy(hist_v, out_hbm.at[wid])

    return hist_kernel(inp)


def _tc_entropy(hists):
    def body(h_ref, o_ref):
        counts = jnp.sum(h_ref[...], axis=0, keepdims=True).astype(jnp.float32)
        counts = counts + LOSS_EPS
        p = counts / jnp.sum(counts)
        o_ref[...] = -jnp.sum(p * jnp.log(p), axis=1, keepdims=True)

    return pl.pallas_call(
        body,
        out_shape=jax.ShapeDtypeStruct((1, 1), jnp.float32),
    )(hists)


@jax.jit
def kernel(input):
    hists = _sc_histogram(input)
    return _tc_entropy(hists)[0, 0]


# speedup vs baseline: 1.0812x; 1.0812x over previous
import dataclasses
import functools

import jax
import jax.numpy as jnp
from jax import lax
from jax.experimental import pallas as pl
from jax.experimental.pallas import tpu as pltpu
from jax.experimental.pallas import tpu_sc as plsc

NBINS = 8192
LOSS_EPS = 1e-08
NC = 2
NS = 16
L = 16
NW = NC * NS
BLK_ROWS = 16


def _sc_histogram(inp):
    b, r, c = inp.shape
    mesh = plsc.VectorSubcoreMesh(core_axis_name="c", subcore_axis_name="s")
    cp = pltpu.CompilerParams(use_tc_tiling_on_sc=True)
    if "needs_layout_passes" in pltpu.CompilerParams.__dataclass_fields__:
        cp = dataclasses.replace(cp, needs_layout_passes=False)

    @functools.partial(
        pl.kernel,
        out_type=jax.ShapeDtypeStruct((NW, NBINS), jnp.int32),
        mesh=mesh,
        scratch_types=[pltpu.VMEM((NBINS,), jnp.int32)],
        compiler_params=cp,
    )
    def hist_kernel(idx_hbm, out_hbm, hist_v):
        wid = lax.axis_index("s") * NC + lax.axis_index("c")

        @pl.loop(0, NBINS, step=L)
        def _(i):
            hist_v[pl.ds(i, L)] = jnp.zeros((L,), jnp.int32)

        ones = jnp.ones((L,), jnp.int32)

        def body(idx_vmem):
            @pl.loop(0, BLK_ROWS, step=2)
            def _(row):
                @pl.loop(0, c, step=16 * L)
                def _(col):
                    xs = [
                        idx_vmem[0, row + rr, pl.ds(col + u * L, L)]
                        for rr in range(2)
                        for u in range(16)
                    ]
                    for x in xs:
                        plsc.addupdate_scatter(hist_v, [x], ones)

        pltpu.emit_pipeline(
            body,
            grid=(b, r // BLK_ROWS),
            in_specs=[
                pl.BlockSpec((1, BLK_ROWS, c), lambda i, j: (i, j, 0))
            ],
            out_specs=[],
            core_axis_name=("c", "s"),
            dimension_semantics=(pltpu.PARALLEL, pltpu.PARALLEL),
        )(idx_hbm)

        pltpu.sync_copy(hist_v, out_hbm.at[wid])

    return hist_kernel(inp)


def _tc_entropy(hists):
    def body(h_ref, o_ref):
        counts = jnp.sum(h_ref[...], axis=0, keepdims=True).astype(jnp.float32)
        counts = counts + LOSS_EPS
        p = counts / jnp.sum(counts)
        o_ref[...] = -jnp.sum(p * jnp.log(p), axis=1, keepdims=True)

    return pl.pallas_call(
        body,
        out_shape=jax.ShapeDtypeStruct((1, 1), jnp.float32),
    )(hists)


@jax.jit
def kernel(input):
    hists = _sc_histogram(input)
    return _tc_entropy(hists)[0, 0]
